# baseline (speedup 1.0000x reference)
import jax
import jax.numpy as jnp
from jax.experimental import pallas as pl
from jax.experimental.pallas import tpu as pltpu

TOK = 8192
HIDDEN = 768
OUT = 768
CTX = 64
E = 8
HALF = HIDDEN // 2
TILE = 512


def _moe_tile_kernel(x_ref, c_ref, gw_ref, gb_ref, w1_ref, b1_ref,
                     w2_ref, b2_ref, out_ref, probs_ref):
    x = x_ref[...]
    c = c_ref[...]

    logits = jax.lax.dot_general(
        x, gw_ref[...], (((1,), (0,)), ((), ())),
        precision=jax.lax.Precision.DEFAULT,
        preferred_element_type=jnp.float32) + gb_ref[...]
    m = jnp.max(logits, axis=-1, keepdims=True)
    ex = jnp.exp(logits - m)
    probs = ex / jnp.sum(ex, axis=-1, keepdims=True)
    probs_ref[...] = probs

    idx = jax.lax.broadcasted_iota(jnp.int32, probs.shape, 1)
    p1 = jnp.max(probs, axis=-1, keepdims=True)
    i1 = jnp.min(jnp.where(probs == p1, idx, E), axis=-1, keepdims=True)
    oh1 = (idx == i1)
    masked = jnp.where(oh1, -jnp.inf, probs)
    p2 = jnp.max(masked, axis=-1, keepdims=True)
    i2 = jnp.min(jnp.where(masked == p2, idx, E), axis=-1, keepdims=True)
    oh2 = (idx == i2)
    denom = p1 + p2 + 1e-8
    coef = (jnp.where(oh1, probs, 0.0) + jnp.where(oh2, probs, 0.0)) / denom

    xc = jnp.concatenate([x, c], axis=1)
    h = jax.lax.dot_general(
        xc, w1_ref[...], (((1,), (0,)), ((), ())),
        precision=jax.lax.Precision.DEFAULT,
        preferred_element_type=jnp.float32) + b1_ref[...]
    je = jax.lax.broadcasted_iota(jnp.int32, (E, E * HALF), 1)
    ee = jax.lax.broadcasted_iota(jnp.int32, (E, E * HALF), 0) * HALF
    mexp = ((je >= ee) & (je < ee + HALF)).astype(jnp.float32)
    coefx = jax.lax.dot_general(
        coef, mexp, (((1,), (0,)), ((), ())),
        precision=jax.lax.Precision.DEFAULT,
        preferred_element_type=jnp.float32)
    hb = jnp.maximum(h, 0.0) * coefx
    acc = jax.lax.dot_general(
        hb, w2_ref[...], (((1,), (0,)), ((), ())),
        precision=jax.lax.Precision.DEFAULT,
        preferred_element_type=jnp.float32)
    acc = acc + jax.lax.dot_general(
        coef, b2_ref[...], (((1,), (0,)), ((), ())),
        precision=jax.lax.Precision.DEFAULT,
        preferred_element_type=jnp.float32)
    out_ref[...] = acc


def kernel(hidden_state, context, gate_W, gate_b, W1, b1, W2, b2):
    w1all = jnp.transpose(W1, (1, 0, 2)).reshape(HIDDEN + CTX, E * HALF)
    w2b = W2.reshape(E * HALF, OUT)
    b1f = b1.reshape(1, E * HALF)
    gb = gate_b.reshape(1, E)

    grid = (TOK // TILE,)
    out_shapes = (
        jax.ShapeDtypeStruct((TOK, OUT), jnp.float32),
        jax.ShapeDtypeStruct((TOK, E), jnp.float32),
    )
    full = lambda *dims: pl.BlockSpec(dims, lambda i: (0,) * len(dims))
    final_out, gate_probs = pl.pallas_call(
        _moe_tile_kernel,
        grid=grid,
        in_specs=[
            pl.BlockSpec((TILE, HIDDEN), lambda i: (i, 0)),
            pl.BlockSpec((TILE, CTX), lambda i: (i, 0)),
            full(HIDDEN, E),
            full(1, E),
            full(HIDDEN + CTX, E * HALF),
            full(1, E * HALF),
            full(E * HALF, OUT),
            full(E, OUT),
        ],
        out_specs=(
            pl.BlockSpec((TILE, OUT), lambda i: (i, 0)),
            pl.BlockSpec((TILE, E), lambda i: (i, 0)),
        ),
        out_shape=out_shapes,
        compiler_params=pltpu.CompilerParams(
            dimension_semantics=("arbitrary",),
        ),
    )(hidden_state, context, gate_W, gb, w1all, b1f, w2b, b2)
    return (final_out, gate_probs)


# speedup vs baseline: 1.4333x; 1.4333x over previous
import jax
import jax.numpy as jnp
from jax.experimental import pallas as pl
from jax.experimental.pallas import tpu as pltpu

TOK = 8192
HIDDEN = 768
OUT = 768
CTX = 64
E = 8
HALF = HIDDEN // 2
TILE = 512


def _moe_tile_kernel(x_ref, c_ref, gw_ref, gb_ref, w1_ref, b1_ref,
                     w2_ref, b2_ref, out_ref, probs_ref, w1s_ref, w2s_ref):
    @pl.when(pl.program_id(0) == 0)
    def _prep():
        for e in range(E):
            w1s_ref[:, e * HALF:(e + 1) * HALF] = (
                w1_ref[e].astype(jnp.bfloat16))
        w2s_ref[...] = w2_ref[...].astype(jnp.bfloat16)

    x = x_ref[...]
    c = c_ref[...]

    logits = jax.lax.dot_general(
        x, gw_ref[...], (((1,), (0,)), ((), ())),
        precision=jax.lax.Precision.DEFAULT,
        preferred_element_type=jnp.float32) + gb_ref[...]
    m = jnp.max(logits, axis=-1, keepdims=True)
    ex = jnp.exp(logits - m)
    probs = ex / jnp.sum(ex, axis=-1, keepdims=True)
    probs_ref[...] = probs

    idx = jax.lax.broadcasted_iota(jnp.int32, probs.shape, 1)
    p1 = jnp.max(probs, axis=-1, keepdims=True)
    i1 = jnp.min(jnp.where(probs == p1, idx, E), axis=-1, keepdims=True)
    oh1 = (idx == i1)
    masked = jnp.where(oh1, -jnp.inf, probs)
    p2 = jnp.max(masked, axis=-1, keepdims=True)
    i2 = jnp.min(jnp.where(masked == p2, idx, E), axis=-1, keepdims=True)
    oh2 = (idx == i2)
    denom = p1 + p2 + 1e-8
    coef = (jnp.where(oh1, probs, 0.0) + jnp.where(oh2, probs, 0.0)) / denom

    xc = jnp.concatenate(
        [x.astype(jnp.bfloat16), c.astype(jnp.bfloat16)], axis=1)
    h = jax.lax.dot_general(
        xc, w1s_ref[...], (((1,), (0,)), ((), ())),
        preferred_element_type=jnp.float32) + b1_ref[...]
    je = jax.lax.broadcasted_iota(jnp.int32, (E, E * HALF), 1)
    ee = jax.lax.broadcasted_iota(jnp.int32, (E, E * HALF), 0) * HALF
    mexp = ((je >= ee) & (je < ee + HALF)).astype(jnp.bfloat16)
    coefx = jax.lax.dot_general(
        coef.astype(jnp.bfloat16), mexp, (((1,), (0,)), ((), ())),
        preferred_element_type=jnp.float32)
    hb = (jnp.maximum(h, 0.0) * coefx).astype(jnp.bfloat16)
    acc = jax.lax.dot_general(
        hb, w2s_ref[...], (((1,), (0,)), ((), ())),
        preferred_element_type=jnp.float32)
    acc = acc + jax.lax.dot_general(
        coef, b2_ref[...], (((1,), (0,)), ((), ())),
        precision=jax.lax.Precision.DEFAULT,
        preferred_element_type=jnp.float32)
    out_ref[...] = acc


def kernel(hidden_state, context, gate_W, gate_b, W1, b1, W2, b2):
    w2f = W2.reshape(E * HALF, OUT)
    b1f = b1.reshape(1, E * HALF)
    gb = gate_b.reshape(1, E)

    grid = (TOK // TILE,)
    out_shapes = (
        jax.ShapeDtypeStruct((TOK, OUT), jnp.float32),
        jax.ShapeDtypeStruct((TOK, E), jnp.float32),
    )
    full = lambda *dims: pl.BlockSpec(dims, lambda i: (0,) * len(dims))
    final_out, gate_probs = pl.pallas_call(
        _moe_tile_kernel,
        grid=grid,
        in_specs=[
            pl.BlockSpec((TILE, HIDDEN), lambda i: (i, 0)),
            pl.BlockSpec((TILE, CTX), lambda i: (i, 0)),
            full(HIDDEN, E),
            full(1, E),
            full(E, HIDDEN + CTX, HALF),
            full(1, E * HALF),
            full(E * HALF, OUT),
            full(E, OUT),
        ],
        out_specs=(
            pl.BlockSpec((TILE, OUT), lambda i: (i, 0)),
            pl.BlockSpec((TILE, E), lambda i: (i, 0)),
        ),
        out_shape=out_shapes,
        scratch_shapes=[
            pltpu.VMEM((HIDDEN + CTX, E * HALF), jnp.bfloat16),
            pltpu.VMEM((E * HALF, OUT), jnp.bfloat16),
        ],
        compiler_params=pltpu.CompilerParams(
            dimension_semantics=("arbitrary",),
        ),
    )(hidden_state, context, gate_W, gb, W1, b1f, w2f, b2)
    return (final_out, gate_probs)


# speedup vs baseline: 1.4598x; 1.0184x over previous
import jax
import jax.numpy as jnp
from jax.experimental import pallas as pl
from jax.experimental.pallas import tpu as pltpu

TOK = 8192
HIDDEN = 768
OUT = 768
CTX = 64
E = 8
HALF = HIDDEN // 2
TILE = 1024


def _moe_tile_kernel(x_ref, c_ref, gw_ref, gb_ref, w1_ref, b1_ref,
                     w2_ref, b2_ref, out_ref, probs_ref, w1s_ref, w2s_ref):
    @pl.when(pl.program_id(0) == 0)
    def _prep():
        for e in range(E):
            w1s_ref[:, e * HALF:(e + 1) * HALF] = (
                w1_ref[e].astype(jnp.bfloat16))
        w2s_ref[...] = w2_ref[...].astype(jnp.bfloat16)

    x = x_ref[...]
    c = c_ref[...]

    logits = jax.lax.dot_general(
        x, gw_ref[...], (((1,), (0,)), ((), ())),
        precision=jax.lax.Precision.DEFAULT,
        preferred_element_type=jnp.float32) + gb_ref[...]
    m = jnp.max(logits, axis=-1, keepdims=True)
    ex = jnp.exp(logits - m)
    probs = ex / jnp.sum(ex, axis=-1, keepdims=True)
    probs_ref[...] = probs

    idx = jax.lax.broadcasted_iota(jnp.int32, probs.shape, 1)
    p1 = jnp.max(probs, axis=-1, keepdims=True)
    i1 = jnp.min(jnp.where(probs == p1, idx, E), axis=-1, keepdims=True)
    oh1 = (idx == i1)
    masked = jnp.where(oh1, -jnp.inf, probs)
    p2 = jnp.max(masked, axis=-1, keepdims=True)
    i2 = jnp.min(jnp.where(masked == p2, idx, E), axis=-1, keepdims=True)
    oh2 = (idx == i2)
    denom = p1 + p2 + 1e-8
    coef = (jnp.where(oh1, probs, 0.0) + jnp.where(oh2, probs, 0.0)) / denom

    xc = jnp.concatenate(
        [x.astype(jnp.bfloat16), c.astype(jnp.bfloat16)], axis=1)
    h = jax.lax.dot_general(
        xc, w1s_ref[...], (((1,), (0,)), ((), ())),
        preferred_element_type=jnp.float32) + b1_ref[...]
    je = jax.lax.broadcasted_iota(jnp.int32, (E, E * HALF), 1)
    ee = jax.lax.broadcasted_iota(jnp.int32, (E, E * HALF), 0) * HALF
    mexp = ((je >= ee) & (je < ee + HALF)).astype(jnp.bfloat16)
    coefx = jax.lax.dot_general(
        coef.astype(jnp.bfloat16), mexp, (((1,), (0,)), ((), ())),
        preferred_element_type=jnp.float32)
    hb = (jnp.maximum(h, 0.0) * coefx).astype(jnp.bfloat16)
    acc = jax.lax.dot_general(
        hb, w2s_ref[...], (((1,), (0,)), ((), ())),
        preferred_element_type=jnp.float32)
    acc = acc + jax.lax.dot_general(
        coef, b2_ref[...], (((1,), (0,)), ((), ())),
        precision=jax.lax.Precision.DEFAULT,
        preferred_element_type=jnp.float32)
    out_ref[...] = acc


def kernel(hidden_state, context, gate_W, gate_b, W1, b1, W2, b2):
    w2f = W2.reshape(E * HALF, OUT)
    b1f = b1.reshape(1, E * HALF)
    gb = gate_b.reshape(1, E)

    grid = (TOK // TILE,)
    out_shapes = (
        jax.ShapeDtypeStruct((TOK, OUT), jnp.float32),
        jax.ShapeDtypeStruct((TOK, E), jnp.float32),
    )
    full = lambda *dims: pl.BlockSpec(dims, lambda i: (0,) * len(dims))
    final_out, gate_probs = pl.pallas_call(
        _moe_tile_kernel,
        grid=grid,
        in_specs=[
            pl.BlockSpec((TILE, HIDDEN), lambda i: (i, 0)),
            pl.BlockSpec((TILE, CTX), lambda i: (i, 0)),
            full(HIDDEN, E),
            full(1, E),
            full(E, HIDDEN + CTX, HALF),
            full(1, E * HALF),
            full(E * HALF, OUT),
            full(E, OUT),
        ],
        out_specs=(
            pl.BlockSpec((TILE, OUT), lambda i: (i, 0)),
            pl.BlockSpec((TILE, E), lambda i: (i, 0)),
        ),
        out_shape=out_shapes,
        scratch_shapes=[
            pltpu.VMEM((HIDDEN + CTX, E * HALF), jnp.bfloat16),
            pltpu.VMEM((E * HALF, OUT), jnp.bfloat16),
        ],
        compiler_params=pltpu.CompilerParams(
            dimension_semantics=("arbitrary",),
        ),
    )(hidden_state, context, gate_W, gb, W1, b1f, w2f, b2)
    return (final_out, gate_probs)
